# Initial kernel scaffold; baseline (speedup 1.0000x reference)
import jax
import jax.numpy as jnp
from jax.experimental import pallas as pl


def kernel(x, W, bias):
    raise NotImplementedError("write your pallas kernel here")


# speedup vs baseline: 3.1257x; 3.1257x over previous
import functools

import jax
import jax.numpy as jnp
from jax.experimental import pallas as pl

DIM = 768
E = 64
K = 8
BT = 1024


def _body(x_ref, w_ref, b_ref, wout_ref, iout_ref, f_ref, p_ref, *, t_total, nsteps):
    step = pl.program_id(0)

    xb = x_ref[...]
    w = w_ref[...]
    scoresT = jax.lax.dot_general(
        w, xb, (((1,), (1,)), ((), ())), preferred_element_type=jnp.float32)

    m = jnp.max(scoresT, axis=0, keepdims=True)
    ex = jnp.exp(scoresT - m)
    probsT = ex / jnp.sum(ex, axis=0, keepdims=True)

    biasedT = probsT + b_ref[...]

    iotaF = jax.lax.broadcasted_iota(jnp.int32, (E, BT), 0).astype(jnp.float32)
    work = biasedT
    wcols = []
    icols = []
    sel_total = jnp.zeros((E, BT), dtype=jnp.float32)
    for _ in range(K):
        cur = jnp.max(work, axis=0, keepdims=True)
        t = jnp.where(work == cur, iotaF, float(E))
        idxF = jnp.min(t, axis=0, keepdims=True)
        onehot = (iotaF == idxF).astype(jnp.float32)
        wcols.append(jnp.sum(onehot * probsT, axis=0, keepdims=True))
        icols.append(idxF)
        sel_total = sel_total + onehot
        work = work - onehot * 3.0e38

    wT = jnp.concatenate(wcols, axis=0)
    iT = jnp.concatenate(icols, axis=0).astype(jnp.int32)
    wout_ref[...] = wT.T
    iout_ref[...] = iT.T

    counts = jnp.sum(sel_total, axis=1)
    psum = jnp.sum(probsT, axis=1)

    @pl.when(step == 0)
    def _init():
        f_ref[...] = jnp.zeros_like(f_ref)
        p_ref[...] = jnp.zeros_like(p_ref)

    f_ref[...] += counts[None, :]
    p_ref[...] += psum[None, :]

    @pl.when(step == nsteps - 1)
    def _fin():
        f_ref[...] = f_ref[...] * (E / (K * t_total + 1e-06))
        p_ref[...] = p_ref[...] / t_total


def kernel(x, W, bias):
    t_total, dim = x.shape
    assert dim == DIM and W.shape == (E, DIM)
    nsteps = t_total // BT
    b2 = bias.reshape(E, 1)

    grid = (nsteps,)
    out_shapes = (
        jax.ShapeDtypeStruct((t_total, K), jnp.float32),
        jax.ShapeDtypeStruct((t_total, K), jnp.int32),
        jax.ShapeDtypeStruct((1, E), jnp.float32),
        jax.ShapeDtypeStruct((1, E), jnp.float32),
    )
    in_specs = [
        pl.BlockSpec((BT, DIM), lambda i: (i, 0)),
        pl.BlockSpec((E, DIM), lambda i: (0, 0)),
        pl.BlockSpec((E, 1), lambda i: (0, 0)),
    ]
    out_specs = (
        pl.BlockSpec((BT, K), lambda i: (i, 0)),
        pl.BlockSpec((BT, K), lambda i: (i, 0)),
        pl.BlockSpec((1, E), lambda i: (0, 0)),
        pl.BlockSpec((1, E), lambda i: (0, 0)),
    )
    weights, indices, f_i, eprobs = pl.pallas_call(
        functools.partial(_body, t_total=t_total, nsteps=nsteps),
        grid=grid,
        in_specs=in_specs,
        out_specs=out_specs,
        out_shape=out_shapes,
    )(x, W, b2)
    return weights, indices, f_i.reshape(E), eprobs.reshape(E)


# speedup vs baseline: 3.6483x; 1.1672x over previous
import functools

import jax
import jax.numpy as jnp
from jax.experimental import pallas as pl

DIM = 768
E = 64
K = 8
BT = 2048


def _body(x_ref, w_ref, b_ref, wout_ref, iout_ref, f_ref, p_ref, *, t_total, nsteps):
    step = pl.program_id(0)

    xb = x_ref[...]
    w = w_ref[...]
    scoresT = jax.lax.dot_general(
        w, xb, (((1,), (1,)), ((), ())), preferred_element_type=jnp.float32)

    m = jnp.max(scoresT, axis=0, keepdims=True)
    ex = jnp.exp(scoresT - m)
    probsT = ex / jnp.sum(ex, axis=0, keepdims=True)

    biasedT = probsT + b_ref[...]

    iotaF = jax.lax.broadcasted_iota(jnp.int32, (E, BT), 0).astype(jnp.float32)
    work = biasedT
    wcols = []
    icols = []
    sel_total = jnp.zeros((E, BT), dtype=jnp.float32)
    for _ in range(K):
        cur = jnp.max(work, axis=0, keepdims=True)
        t = jnp.where(work == cur, iotaF, float(E))
        idxF = jnp.min(t, axis=0, keepdims=True)
        onehot = (iotaF == idxF).astype(jnp.float32)
        wcols.append(jnp.sum(onehot * probsT, axis=0, keepdims=True))
        icols.append(idxF)
        sel_total = sel_total + onehot
        work = work - onehot * 3.0e38

    wT = jnp.concatenate(wcols, axis=0)
    iT = jnp.concatenate(icols, axis=0).astype(jnp.int32)
    wout_ref[...] = wT.T
    iout_ref[...] = iT.T

    counts = jnp.sum(sel_total, axis=1)
    psum = jnp.sum(probsT, axis=1)

    @pl.when(step == 0)
    def _init():
        f_ref[...] = jnp.zeros_like(f_ref)
        p_ref[...] = jnp.zeros_like(p_ref)

    f_ref[...] += counts[None, :]
    p_ref[...] += psum[None, :]

    @pl.when(step == nsteps - 1)
    def _fin():
        f_ref[...] = f_ref[...] * (E / (K * t_total + 1e-06))
        p_ref[...] = p_ref[...] / t_total


def kernel(x, W, bias):
    t_total, dim = x.shape
    assert dim == DIM and W.shape == (E, DIM)
    nsteps = t_total // BT
    b2 = bias.reshape(E, 1)

    grid = (nsteps,)
    out_shapes = (
        jax.ShapeDtypeStruct((t_total, K), jnp.float32),
        jax.ShapeDtypeStruct((t_total, K), jnp.int32),
        jax.ShapeDtypeStruct((1, E), jnp.float32),
        jax.ShapeDtypeStruct((1, E), jnp.float32),
    )
    in_specs = [
        pl.BlockSpec((BT, DIM), lambda i: (i, 0)),
        pl.BlockSpec((E, DIM), lambda i: (0, 0)),
        pl.BlockSpec((E, 1), lambda i: (0, 0)),
    ]
    out_specs = (
        pl.BlockSpec((BT, K), lambda i: (i, 0)),
        pl.BlockSpec((BT, K), lambda i: (i, 0)),
        pl.BlockSpec((1, E), lambda i: (0, 0)),
        pl.BlockSpec((1, E), lambda i: (0, 0)),
    )
    weights, indices, f_i, eprobs = pl.pallas_call(
        functools.partial(_body, t_total=t_total, nsteps=nsteps),
        grid=grid,
        in_specs=in_specs,
        out_specs=out_specs,
        out_shape=out_shapes,
    )(x, W, b2)
    return weights, indices, f_i.reshape(E), eprobs.reshape(E)


# speedup vs baseline: 3.8139x; 1.0454x over previous
import functools

import jax
import jax.numpy as jnp
from jax.experimental import pallas as pl

DIM = 768
E = 64
K = 8
BT = 4096


def _body(x_ref, w_ref, b_ref, wout_ref, iout_ref, f_ref, p_ref, *, t_total, nsteps):
    step = pl.program_id(0)

    xb = x_ref[...]
    w = w_ref[...]
    scoresT = jax.lax.dot_general(
        w, xb, (((1,), (1,)), ((), ())), preferred_element_type=jnp.float32)

    m = jnp.max(scoresT, axis=0, keepdims=True)
    ex = jnp.exp(scoresT - m)
    probsT = ex / jnp.sum(ex, axis=0, keepdims=True)

    biasedT = probsT + b_ref[...]

    iotaF = jax.lax.broadcasted_iota(jnp.int32, (E, BT), 0).astype(jnp.float32)
    work = biasedT
    wcols = []
    icols = []
    sel_total = jnp.zeros((E, BT), dtype=jnp.float32)
    for _ in range(K):
        cur = jnp.max(work, axis=0, keepdims=True)
        t = jnp.where(work == cur, iotaF, float(E))
        idxF = jnp.min(t, axis=0, keepdims=True)
        onehot = (iotaF == idxF).astype(jnp.float32)
        wcols.append(jnp.sum(onehot * probsT, axis=0, keepdims=True))
        icols.append(idxF)
        sel_total = sel_total + onehot
        work = work - onehot * 3.0e38

    wT = jnp.concatenate(wcols, axis=0)
    iT = jnp.concatenate(icols, axis=0).astype(jnp.int32)
    wout_ref[...] = wT.T
    iout_ref[...] = iT.T

    counts = jnp.sum(sel_total, axis=1)
    psum = jnp.sum(probsT, axis=1)

    @pl.when(step == 0)
    def _init():
        f_ref[...] = jnp.zeros_like(f_ref)
        p_ref[...] = jnp.zeros_like(p_ref)

    f_ref[...] += counts[None, :]
    p_ref[...] += psum[None, :]

    @pl.when(step == nsteps - 1)
    def _fin():
        f_ref[...] = f_ref[...] * (E / (K * t_total + 1e-06))
        p_ref[...] = p_ref[...] / t_total


def kernel(x, W, bias):
    t_total, dim = x.shape
    assert dim == DIM and W.shape == (E, DIM)
    nsteps = t_total // BT
    b2 = bias.reshape(E, 1)

    grid = (nsteps,)
    out_shapes = (
        jax.ShapeDtypeStruct((t_total, K), jnp.float32),
        jax.ShapeDtypeStruct((t_total, K), jnp.int32),
        jax.ShapeDtypeStruct((1, E), jnp.float32),
        jax.ShapeDtypeStruct((1, E), jnp.float32),
    )
    in_specs = [
        pl.BlockSpec((BT, DIM), lambda i: (i, 0)),
        pl.BlockSpec((E, DIM), lambda i: (0, 0)),
        pl.BlockSpec((E, 1), lambda i: (0, 0)),
    ]
    out_specs = (
        pl.BlockSpec((BT, K), lambda i: (i, 0)),
        pl.BlockSpec((BT, K), lambda i: (i, 0)),
        pl.BlockSpec((1, E), lambda i: (0, 0)),
        pl.BlockSpec((1, E), lambda i: (0, 0)),
    )
    weights, indices, f_i, eprobs = pl.pallas_call(
        functools.partial(_body, t_total=t_total, nsteps=nsteps),
        grid=grid,
        in_specs=in_specs,
        out_specs=out_specs,
        out_shape=out_shapes,
    )(x, W, b2)
    return weights, indices, f_i.reshape(E), eprobs.reshape(E)
